# initial kernel scaffold (unmeasured)
import jax
import jax.numpy as jnp
from jax import lax
from jax.experimental import pallas as pl
from jax.experimental.pallas import tpu as pltpu

N_DEV = 16


def kernel(A, B):
    M, K = A.shape
    _, N = B.shape
    CH = M // N_DEV

    def body(a_ref, b_ref, out_ref, acc_ref, comm_ref, send_sems, recv_sems):
        my = lax.axis_index("i")
        left = (my - 1) % N_DEV
        right = (my + 1) % N_DEV

        barrier_sem = pltpu.get_barrier_semaphore()
        for nbr in (left, right):
            pl.semaphore_signal(
                barrier_sem, inc=1,
                device_id=(nbr,), device_id_type=pl.DeviceIdType.MESH,
            )
        pl.semaphore_wait(barrier_sem, 2)

        acc_ref[:, :] = jnp.dot(
            a_ref[:, :], b_ref[:, :], preferred_element_type=jnp.float32
        )

        comm_ref[0, :, :] = acc_ref[pl.ds(my * CH, CH), :]
        for s in range(N_DEV - 1):
            send_slot = s % 2
            recv_slot = (s + 1) % 2
            rdma = pltpu.make_async_remote_copy(
                src_ref=comm_ref.at[send_slot],
                dst_ref=comm_ref.at[recv_slot],
                send_sem=send_sems.at[send_slot],
                recv_sem=recv_sems.at[recv_slot],
                device_id=(right,),
                device_id_type=pl.DeviceIdType.MESH,
            )
            rdma.start()
            rdma.wait()
            recv_idx = (my - s - 1) % N_DEV
            comm_ref[recv_slot, :, :] = (
                comm_ref[recv_slot, :, :] + acc_ref[pl.ds(recv_idx * CH, CH), :]
            )

        own = (my + 1) % N_DEV
        out_ref[pl.ds(own * CH, CH), :] = comm_ref[(N_DEV - 1) % 2, :, :]

        for t in range(N_DEV - 1):
            step = (N_DEV - 1) + t
            send_slot = step % 2
            recv_slot = (step + 1) % 2
            rdma = pltpu.make_async_remote_copy(
                src_ref=comm_ref.at[send_slot],
                dst_ref=comm_ref.at[recv_slot],
                send_sem=send_sems.at[send_slot],
                recv_sem=recv_sems.at[recv_slot],
                device_id=(right,),
                device_id_type=pl.DeviceIdType.MESH,
            )
            rdma.start()
            rdma.wait()
            recv_idx = (my - t) % N_DEV
            out_ref[pl.ds(recv_idx * CH, CH), :] = comm_ref[recv_slot, :, :]

    return pl.pallas_call(
        body,
        out_shape=jax.ShapeDtypeStruct((M, N), jnp.float32),
        in_specs=[
            pl.BlockSpec(memory_space=pltpu.VMEM),
            pl.BlockSpec(memory_space=pltpu.VMEM),
        ],
        out_specs=pl.BlockSpec(memory_space=pltpu.VMEM),
        scratch_shapes=[
            pltpu.VMEM((M, N), jnp.float32),
            pltpu.VMEM((2, CH, N), jnp.float32),
            pltpu.SemaphoreType.DMA((2,)),
            pltpu.SemaphoreType.DMA((2,)),
        ],
        compiler_params=pltpu.CompilerParams(collective_id=0),
    )(A, B)


# baseline (device time: 431524 ns/iter reference)
import jax
import jax.numpy as jnp
from jax import lax
from jax.experimental import pallas as pl
from jax.experimental.pallas import tpu as pltpu

N_DEV = 16


def kernel(A, B):
    M, K = A.shape
    _, N = B.shape
    CH = M // N_DEV

    def body(a_ref, b_ref, out_ref, acc_ref, comm_ref, send_sems, recv_sems):
        my = lax.axis_index("i")
        left = (my - 1) % N_DEV
        right = (my + 1) % N_DEV

        barrier_sem = pltpu.get_barrier_semaphore()
        for nbr in (left, right):
            pl.semaphore_signal(
                barrier_sem, inc=1,
                device_id=(nbr,), device_id_type=pl.DeviceIdType.MESH,
            )
        pl.semaphore_wait(barrier_sem, 2)

        acc_ref[:, :] = jnp.dot(
            a_ref[:, :], b_ref[:, :], preferred_element_type=jnp.float32
        )

        comm_ref[0, :, :] = acc_ref[pl.ds(my * CH, CH), :]
        for s in range(N_DEV - 1):
            send_slot = s % 2
            recv_slot = (s + 1) % 2
            rdma = pltpu.make_async_remote_copy(
                src_ref=comm_ref.at[send_slot],
                dst_ref=comm_ref.at[recv_slot],
                send_sem=send_sems.at[send_slot],
                recv_sem=recv_sems.at[recv_slot],
                device_id=(right,),
                device_id_type=pl.DeviceIdType.MESH,
            )
            rdma.start()
            rdma.wait()
            recv_idx = (my - s - 1) % N_DEV
            comm_ref[recv_slot, :, :] = (
                comm_ref[recv_slot, :, :] + acc_ref[pl.ds(recv_idx * CH, CH), :]
            )

        own = (my + 1) % N_DEV
        out_ref[pl.ds(own * CH, CH), :] = comm_ref[(N_DEV - 1) % 2, :, :]

        for t in range(N_DEV - 1):
            step = (N_DEV - 1) + t
            send_slot = step % 2
            recv_slot = (step + 1) % 2
            rdma = pltpu.make_async_remote_copy(
                src_ref=comm_ref.at[send_slot],
                dst_ref=comm_ref.at[recv_slot],
                send_sem=send_sems.at[send_slot],
                recv_sem=recv_sems.at[recv_slot],
                device_id=(right,),
                device_id_type=pl.DeviceIdType.MESH,
            )
            rdma.start()
            rdma.wait()
            recv_idx = (my - t) % N_DEV
            out_ref[pl.ds(recv_idx * CH, CH), :] = comm_ref[recv_slot, :, :]

    return pl.pallas_call(
        body,
        out_shape=jax.ShapeDtypeStruct((M, N), jnp.float32),
        in_specs=[
            pl.BlockSpec(memory_space=pltpu.VMEM),
            pl.BlockSpec(memory_space=pltpu.VMEM),
        ],
        out_specs=pl.BlockSpec(memory_space=pltpu.VMEM),
        scratch_shapes=[
            pltpu.VMEM((M, N), jnp.float32),
            pltpu.VMEM((2, CH, N), jnp.float32),
            pltpu.SemaphoreType.DMA((2,)),
            pltpu.SemaphoreType.DMA((2,)),
        ],
        compiler_params=pltpu.CompilerParams(
            collective_id=0,
            vmem_limit_bytes=64 * 1024 * 1024,
        ),
    )(A, B)


# device time: 302943 ns/iter; 1.4244x vs baseline; 1.4244x over previous
import jax
import jax.numpy as jnp
from jax import lax
from jax.experimental import pallas as pl
from jax.experimental.pallas import tpu as pltpu

N_DEV = 16


def kernel(A, B):
    M, K = A.shape
    _, N = B.shape
    CH = M // N_DEV
    H = N // 2

    def body(a_ref, b_ref, out_ref, acc_ref,
             comm_f, comm_b, send_f, recv_f, send_b, recv_b):
        my = lax.axis_index("i")
        left = (my - 1) % N_DEV
        right = (my + 1) % N_DEV

        barrier_sem = pltpu.get_barrier_semaphore()
        for nbr in (left, right):
            pl.semaphore_signal(
                barrier_sem, inc=1,
                device_id=(nbr,), device_id_type=pl.DeviceIdType.MESH,
            )
        pl.semaphore_wait(barrier_sem, 2)

        acc_ref[:, :] = jnp.dot(
            a_ref[:, :], b_ref[:, :], preferred_element_type=jnp.float32
        )

        def hop(src_comm, dst_comm, s_sems, r_sems, slot_s, slot_r, dst):
            rdma = pltpu.make_async_remote_copy(
                src_ref=src_comm.at[slot_s],
                dst_ref=dst_comm.at[slot_r],
                send_sem=s_sems.at[slot_s],
                recv_sem=r_sems.at[slot_r],
                device_id=(dst,),
                device_id_type=pl.DeviceIdType.MESH,
            )
            rdma.start()
            return rdma

        comm_f[0, :, :] = acc_ref[pl.ds(my * CH, CH), 0:H]
        comm_b[0, :, :] = acc_ref[pl.ds(my * CH, CH), H:N]
        for s in range(N_DEV - 1):
            ss = s % 2
            rs = (s + 1) % 2
            rf = hop(comm_f, comm_f, send_f, recv_f, ss, rs, right)
            rb = hop(comm_b, comm_b, send_b, recv_b, ss, rs, left)
            rf.wait()
            rb.wait()
            idx_f = (my - s - 1) % N_DEV
            idx_b = (my + s + 1) % N_DEV
            comm_f[rs, :, :] = (
                comm_f[rs, :, :] + acc_ref[pl.ds(idx_f * CH, CH), 0:H]
            )
            comm_b[rs, :, :] = (
                comm_b[rs, :, :] + acc_ref[pl.ds(idx_b * CH, CH), H:N]
            )

        last = (N_DEV - 1) % 2
        own_f = (my + 1) % N_DEV
        own_b = (my - 1) % N_DEV
        out_ref[pl.ds(own_f * CH, CH), 0:H] = comm_f[last, :, :]
        out_ref[pl.ds(own_b * CH, CH), H:N] = comm_b[last, :, :]

        for t in range(N_DEV - 1):
            step = (N_DEV - 1) + t
            ss = step % 2
            rs = (step + 1) % 2
            rf = hop(comm_f, comm_f, send_f, recv_f, ss, rs, right)
            rb = hop(comm_b, comm_b, send_b, recv_b, ss, rs, left)
            rf.wait()
            rb.wait()
            idx_f = (my - t) % N_DEV
            idx_b = (my + t) % N_DEV
            out_ref[pl.ds(idx_f * CH, CH), 0:H] = comm_f[rs, :, :]
            out_ref[pl.ds(idx_b * CH, CH), H:N] = comm_b[rs, :, :]

    return pl.pallas_call(
        body,
        out_shape=jax.ShapeDtypeStruct((M, N), jnp.float32),
        in_specs=[
            pl.BlockSpec(memory_space=pltpu.VMEM),
            pl.BlockSpec(memory_space=pltpu.VMEM),
        ],
        out_specs=pl.BlockSpec(memory_space=pltpu.VMEM),
        scratch_shapes=[
            pltpu.VMEM((M, N), jnp.float32),
            pltpu.VMEM((2, CH, H), jnp.float32),
            pltpu.VMEM((2, CH, H), jnp.float32),
            pltpu.SemaphoreType.DMA((2,)),
            pltpu.SemaphoreType.DMA((2,)),
            pltpu.SemaphoreType.DMA((2,)),
            pltpu.SemaphoreType.DMA((2,)),
        ],
        compiler_params=pltpu.CompilerParams(
            collective_id=0,
            vmem_limit_bytes=64 * 1024 * 1024,
        ),
    )(A, B)


# device time: 254731 ns/iter; 1.6940x vs baseline; 1.1893x over previous
import jax
import jax.numpy as jnp
from jax import lax
from jax.experimental import pallas as pl
from jax.experimental.pallas import tpu as pltpu

N_DEV = 16

RING = [0, 4, 8, 12, 13, 9, 5, 1, 2, 6, 10, 14, 15, 11, 7, 3]
INV_RING = [0] * N_DEV
for _p, _m in enumerate(RING):
    INV_RING[_m] = _p


def kernel(A, B):
    M, K = A.shape
    _, N = B.shape
    CH = M // N_DEV
    H = N // 2

    ring = jnp.asarray(RING, dtype=jnp.int32)
    inv = jnp.asarray(INV_RING, dtype=jnp.int32)
    m = lax.axis_index("i").astype(jnp.int32)
    r = inv[m]
    right = ring[(r + 1) % N_DEV]
    left = ring[(r - 1) % N_DEV]
    scalars = [jnp.reshape(v, (1,)) for v in (r, left, right)]

    def body(r_ref, left_ref, right_ref, a_ref, b_ref, out_ref,
             comm_f, comm_b, pf, pb, send_f, recv_f, send_b, recv_b):
        r = r_ref[0]
        left = left_ref[0]
        right = right_ref[0]

        barrier_sem = pltpu.get_barrier_semaphore()
        for nbr in (left, right):
            pl.semaphore_signal(
                barrier_sem, inc=1,
                device_id=(nbr,), device_id_type=pl.DeviceIdType.MESH,
            )
        pl.semaphore_wait(barrier_sem, 2)

        def hop(comm, s_sems, r_sems, slot_s, slot_r, dst):
            rdma = pltpu.make_async_remote_copy(
                src_ref=comm.at[slot_s],
                dst_ref=comm.at[slot_r],
                send_sem=s_sems.at[slot_s],
                recv_sem=r_sems.at[slot_r],
                device_id=(dst,),
                device_id_type=pl.DeviceIdType.MESH,
            )
            rdma.start()
            return rdma

        def stripe(idx, lo, hi):
            return jnp.dot(
                a_ref[pl.ds(idx * CH, CH), :], b_ref[:, lo:hi],
                preferred_element_type=jnp.float32,
            )

        comm_f[0, :, :] = stripe(r, 0, H)
        comm_b[0, :, :] = stripe(r, H, N)
        for s in range(N_DEV - 1):
            ss = s % 2
            rs = (s + 1) % 2
            rf = hop(comm_f, send_f, recv_f, ss, rs, right)
            rb = hop(comm_b, send_b, recv_b, ss, rs, left)
            idx_f = (r - s - 1) % N_DEV
            idx_b = (r + s + 1) % N_DEV
            pf[:, :] = stripe(idx_f, 0, H)
            pb[:, :] = stripe(idx_b, H, N)
            rf.wait()
            rb.wait()
            comm_f[rs, :, :] = comm_f[rs, :, :] + pf[:, :]
            comm_b[rs, :, :] = comm_b[rs, :, :] + pb[:, :]

        for t in range(N_DEV - 1):
            step = (N_DEV - 1) + t
            ss = step % 2
            rs = (step + 1) % 2
            rf = hop(comm_f, send_f, recv_f, ss, rs, right)
            rb = hop(comm_b, send_b, recv_b, ss, rs, left)
            idx_f = (r + 1 - t) % N_DEV
            idx_b = (r - 1 + t) % N_DEV
            out_ref[pl.ds(idx_f * CH, CH), 0:H] = comm_f[ss, :, :]
            out_ref[pl.ds(idx_b * CH, CH), H:N] = comm_b[ss, :, :]
            rf.wait()
            rb.wait()
        last = (2 * (N_DEV - 1)) % 2
        idx_f = (r + 2) % N_DEV
        idx_b = (r - 2) % N_DEV
        out_ref[pl.ds(idx_f * CH, CH), 0:H] = comm_f[last, :, :]
        out_ref[pl.ds(idx_b * CH, CH), H:N] = comm_b[last, :, :]

    return pl.pallas_call(
        body,
        out_shape=jax.ShapeDtypeStruct((M, N), jnp.float32),
        in_specs=[
            pl.BlockSpec(memory_space=pltpu.SMEM),
            pl.BlockSpec(memory_space=pltpu.SMEM),
            pl.BlockSpec(memory_space=pltpu.SMEM),
            pl.BlockSpec(memory_space=pltpu.VMEM),
            pl.BlockSpec(memory_space=pltpu.VMEM),
        ],
        out_specs=pl.BlockSpec(memory_space=pltpu.VMEM),
        scratch_shapes=[
            pltpu.VMEM((2, CH, H), jnp.float32),
            pltpu.VMEM((2, CH, H), jnp.float32),
            pltpu.VMEM((CH, H), jnp.float32),
            pltpu.VMEM((CH, H), jnp.float32),
            pltpu.SemaphoreType.DMA((2,)),
            pltpu.SemaphoreType.DMA((2,)),
            pltpu.SemaphoreType.DMA((2,)),
            pltpu.SemaphoreType.DMA((2,)),
        ],
        compiler_params=pltpu.CompilerParams(
            collective_id=0,
            vmem_limit_bytes=64 * 1024 * 1024,
        ),
    )(*scalars, A, B)


# device time: 204139 ns/iter; 2.1139x vs baseline; 1.2478x over previous
import jax
import jax.numpy as jnp
from jax import lax
from jax.experimental import pallas as pl
from jax.experimental.pallas import tpu as pltpu

N_DEV = 16
NSLOT = 4
KSUB = 2
RT = 2 * (N_DEV - 1)

RING = [0, 4, 8, 12, 13, 9, 5, 1, 2, 6, 10, 14, 15, 11, 7, 3]
INV_RING = [0] * N_DEV
for _p, _m in enumerate(RING):
    INV_RING[_m] = _p


def kernel(A, B):
    M, K = A.shape
    _, N = B.shape
    CH = M // N_DEV
    H = N // 2
    CHK = CH // KSUB
    NSUB = NSLOT * KSUB
    SEED = NSUB

    ring = jnp.asarray(RING, dtype=jnp.int32)
    inv = jnp.asarray(INV_RING, dtype=jnp.int32)
    m = lax.axis_index("i").astype(jnp.int32)
    r = inv[m]
    right = ring[(r + 1) % N_DEV]
    left = ring[(r - 1) % N_DEV]
    scalars = [jnp.reshape(v, (1,)) for v in (r, left, right)]

    def body(r_ref, left_ref, right_ref, a_ref, b_ref, out_ref,
             comm_f, comm_b, b_bf, pf, pb,
             send_f, recv_f, send_b, recv_b, credit_f, credit_b):
        r = r_ref[0]
        left = left_ref[0]
        right = right_ref[0]

        def stripe(idx, lo, hi):
            a_blk = a_ref[pl.ds(idx * CH, CH), :].astype(jnp.bfloat16)
            return jnp.dot(
                a_blk, b_bf[:, lo:hi], preferred_element_type=jnp.float32
            )

        def sub(s, j):
            return (s % NSLOT) * KSUB + j

        def mk(fwd, s, j):
            comm = comm_f if fwd else comm_b
            ssem = send_f if fwd else send_b
            rsem = recv_f if fwd else recv_b
            src_idx = SEED + j if s == 0 else sub(s - 1, j)
            return pltpu.make_async_remote_copy(
                src_ref=comm.at[src_idx],
                dst_ref=comm.at[sub(s, j)],
                send_sem=ssem.at[sub(s, j)],
                recv_sem=rsem.at[sub(s, j)],
                device_id=(right if fwd else left,),
                device_id_type=pl.DeviceIdType.MESH,
            )

        b_bf[:, :] = b_ref[:, :].astype(jnp.bfloat16)
        sf = stripe(r, 0, H)
        sb = stripe(r, H, N)
        for j in range(KSUB):
            comm_f[SEED + j, :, :] = sf[j * CHK:(j + 1) * CHK, :]
            comm_b[SEED + j, :, :] = sb[j * CHK:(j + 1) * CHK, :]
        pf[0, :, :] = stripe((r - 1) % N_DEV, 0, H)
        pb[0, :, :] = stripe((r + 1) % N_DEV, H, N)

        barrier_sem = pltpu.get_barrier_semaphore()
        for nbr in (left, right):
            pl.semaphore_signal(
                barrier_sem, inc=1,
                device_id=(nbr,), device_id_type=pl.DeviceIdType.MESH,
            )
        pl.semaphore_wait(barrier_sem, 2)

        sends_f = {}
        sends_b = {}
        for j in range(KSUB):
            sends_f[(0, j)] = mk(True, 0, j)
            sends_b[(0, j)] = mk(False, 0, j)
            sends_f[(0, j)].start()
            sends_b[(0, j)].start()

        for s in range(RT):
            if s + 1 <= N_DEV - 2:
                pf[(s + 1) % 2, :, :] = stripe((r - s - 2) % N_DEV, 0, H)
                pb[(s + 1) % 2, :, :] = stripe((r + s + 2) % N_DEV, H, N)
            for j in range(KSUB):
                mk(True, s, j).wait_recv()
                mk(False, s, j).wait_recv()
                if s <= N_DEV - 2:
                    comm_f[sub(s, j), :, :] = (
                        comm_f[sub(s, j), :, :]
                        + pf[s % 2, pl.ds(j * CHK, CHK), :]
                    )
                    comm_b[sub(s, j), :, :] = (
                        comm_b[sub(s, j), :, :]
                        + pb[s % 2, pl.ds(j * CHK, CHK), :]
                    )
                else:
                    t = s - (N_DEV - 1)
                    idx_f = (r - t) % N_DEV
                    idx_b = (r + t) % N_DEV
                    out_ref[pl.ds(idx_f * CH + j * CHK, CHK), 0:H] = (
                        comm_f[sub(s, j), :, :]
                    )
                    out_ref[pl.ds(idx_b * CH + j * CHK, CHK), H:N] = (
                        comm_b[sub(s, j), :, :]
                    )
                if s < RT - 1:
                    if j == 0 and s + 1 >= NSLOT:
                        pl.semaphore_wait(credit_f, 1)
                        pl.semaphore_wait(credit_b, 1)
                    if s + 1 >= NSLOT:
                        sends_f[(s + 1 - NSLOT, j)].wait_send()
                        sends_b[(s + 1 - NSLOT, j)].wait_send()
                    nf = mk(True, s + 1, j)
                    nb = mk(False, s + 1, j)
                    nf.start()
                    nb.start()
                    sends_f[(s + 1, j)] = nf
                    sends_b[(s + 1, j)] = nb
            if s == N_DEV - 2:
                own_f = (r + 1) % N_DEV
                own_b = (r - 1) % N_DEV
                for j in range(KSUB):
                    out_ref[pl.ds(own_f * CH + j * CHK, CHK), 0:H] = (
                        comm_f[sub(s, j), :, :]
                    )
                    out_ref[pl.ds(own_b * CH + j * CHK, CHK), H:N] = (
                        comm_b[sub(s, j), :, :]
                    )
            if s <= RT - 1 - NSLOT:
                pl.semaphore_signal(
                    credit_f, inc=1,
                    device_id=(left,), device_id_type=pl.DeviceIdType.MESH,
                )
                pl.semaphore_signal(
                    credit_b, inc=1,
                    device_id=(right,), device_id_type=pl.DeviceIdType.MESH,
                )

        for s in range(RT - NSLOT, RT):
            for j in range(KSUB):
                sends_f[(s, j)].wait_send()
                sends_b[(s, j)].wait_send()

    return pl.pallas_call(
        body,
        out_shape=jax.ShapeDtypeStruct((M, N), jnp.float32),
        in_specs=[
            pl.BlockSpec(memory_space=pltpu.SMEM),
            pl.BlockSpec(memory_space=pltpu.SMEM),
            pl.BlockSpec(memory_space=pltpu.SMEM),
            pl.BlockSpec(memory_space=pltpu.VMEM),
            pl.BlockSpec(memory_space=pltpu.VMEM),
        ],
        out_specs=pl.BlockSpec(memory_space=pltpu.VMEM),
        scratch_shapes=[
            pltpu.VMEM((NSUB + KSUB, CHK, H), jnp.float32),
            pltpu.VMEM((NSUB + KSUB, CHK, H), jnp.float32),
            pltpu.VMEM((K, N), jnp.bfloat16),
            pltpu.VMEM((2, CH, H), jnp.float32),
            pltpu.VMEM((2, CH, H), jnp.float32),
            pltpu.SemaphoreType.DMA((NSUB,)),
            pltpu.SemaphoreType.DMA((NSUB,)),
            pltpu.SemaphoreType.DMA((NSUB,)),
            pltpu.SemaphoreType.DMA((NSUB,)),
            pltpu.SemaphoreType.REGULAR,
            pltpu.SemaphoreType.REGULAR,
        ],
        compiler_params=pltpu.CompilerParams(
            collective_id=0,
            vmem_limit_bytes=64 * 1024 * 1024,
        ),
    )(*scalars, A, B)


# device time: 203845 ns/iter; 2.1169x vs baseline; 1.0014x over previous
import jax
import jax.numpy as jnp
from jax import lax
from jax.experimental import pallas as pl
from jax.experimental.pallas import tpu as pltpu

N_DEV = 16
NSLOT = 4
KSUB = 4
RT = 2 * (N_DEV - 1)

RING = [0, 4, 8, 12, 13, 9, 5, 1, 2, 6, 10, 14, 15, 11, 7, 3]
INV_RING = [0] * N_DEV
for _p, _m in enumerate(RING):
    INV_RING[_m] = _p


def kernel(A, B):
    M, K = A.shape
    _, N = B.shape
    CH = M // N_DEV
    H = N // 2
    CHK = CH // KSUB
    NSUB = NSLOT * KSUB
    SEED = NSUB

    ring = jnp.asarray(RING, dtype=jnp.int32)
    inv = jnp.asarray(INV_RING, dtype=jnp.int32)
    m = lax.axis_index("i").astype(jnp.int32)
    r = inv[m]
    right = ring[(r + 1) % N_DEV]
    left = ring[(r - 1) % N_DEV]
    scalars = [jnp.reshape(v, (1,)) for v in (r, left, right)]

    def body(r_ref, left_ref, right_ref, a_ref, b_ref, out_ref,
             comm_f, comm_b, b_bf, pf, pb,
             send_f, recv_f, send_b, recv_b, credit_f, credit_b):
        r = r_ref[0]
        left = left_ref[0]
        right = right_ref[0]

        def stripe(idx, lo, hi):
            a_blk = a_ref[pl.ds(idx * CH, CH), :].astype(jnp.bfloat16)
            return jnp.dot(
                a_blk, b_bf[:, lo:hi], preferred_element_type=jnp.float32
            )

        def sub(s, j):
            return (s % NSLOT) * KSUB + j

        def mk(fwd, s, j):
            comm = comm_f if fwd else comm_b
            ssem = send_f if fwd else send_b
            rsem = recv_f if fwd else recv_b
            src_idx = SEED + j if s == 0 else sub(s - 1, j)
            return pltpu.make_async_remote_copy(
                src_ref=comm.at[src_idx],
                dst_ref=comm.at[sub(s, j)],
                send_sem=ssem.at[sub(s, j)],
                recv_sem=rsem.at[sub(s, j)],
                device_id=(right if fwd else left,),
                device_id_type=pl.DeviceIdType.MESH,
            )

        b_bf[:, :] = b_ref[:, :].astype(jnp.bfloat16)
        sf = stripe(r, 0, H)
        sb = stripe(r, H, N)
        for j in range(KSUB):
            comm_f[SEED + j, :, :] = sf[j * CHK:(j + 1) * CHK, :]
            comm_b[SEED + j, :, :] = sb[j * CHK:(j + 1) * CHK, :]
        barrier_sem = pltpu.get_barrier_semaphore()
        for nbr in (left, right):
            pl.semaphore_signal(
                barrier_sem, inc=1,
                device_id=(nbr,), device_id_type=pl.DeviceIdType.MESH,
            )
        pl.semaphore_wait(barrier_sem, 2)

        sends_f = {}
        sends_b = {}
        for j in range(KSUB):
            sends_f[(0, j)] = mk(True, 0, j)
            sends_b[(0, j)] = mk(False, 0, j)
            sends_f[(0, j)].start()
            sends_b[(0, j)].start()

        pf[0, :, :] = stripe((r - 1) % N_DEV, 0, H)
        pb[0, :, :] = stripe((r + 1) % N_DEV, H, N)

        for s in range(RT):
            if s + 1 <= N_DEV - 2:
                pf[(s + 1) % 2, :, :] = stripe((r - s - 2) % N_DEV, 0, H)
                pb[(s + 1) % 2, :, :] = stripe((r + s + 2) % N_DEV, H, N)
            for j in range(KSUB):
                mk(True, s, j).wait_recv()
                mk(False, s, j).wait_recv()
                if s <= N_DEV - 2:
                    comm_f[sub(s, j), :, :] = (
                        comm_f[sub(s, j), :, :]
                        + pf[s % 2, pl.ds(j * CHK, CHK), :]
                    )
                    comm_b[sub(s, j), :, :] = (
                        comm_b[sub(s, j), :, :]
                        + pb[s % 2, pl.ds(j * CHK, CHK), :]
                    )
                else:
                    t = s - (N_DEV - 1)
                    idx_f = (r - t) % N_DEV
                    idx_b = (r + t) % N_DEV
                    out_ref[pl.ds(idx_f * CH + j * CHK, CHK), 0:H] = (
                        comm_f[sub(s, j), :, :]
                    )
                    out_ref[pl.ds(idx_b * CH + j * CHK, CHK), H:N] = (
                        comm_b[sub(s, j), :, :]
                    )
                if s < RT - 1:
                    if j == 0 and s + 1 >= NSLOT:
                        pl.semaphore_wait(credit_f, 1)
                        pl.semaphore_wait(credit_b, 1)
                    if s + 1 >= NSLOT:
                        sends_f[(s + 1 - NSLOT, j)].wait_send()
                        sends_b[(s + 1 - NSLOT, j)].wait_send()
                    nf = mk(True, s + 1, j)
                    nb = mk(False, s + 1, j)
                    nf.start()
                    nb.start()
                    sends_f[(s + 1, j)] = nf
                    sends_b[(s + 1, j)] = nb
            if s == N_DEV - 2:
                own_f = (r + 1) % N_DEV
                own_b = (r - 1) % N_DEV
                for j in range(KSUB):
                    out_ref[pl.ds(own_f * CH + j * CHK, CHK), 0:H] = (
                        comm_f[sub(s, j), :, :]
                    )
                    out_ref[pl.ds(own_b * CH + j * CHK, CHK), H:N] = (
                        comm_b[sub(s, j), :, :]
                    )
            if s <= RT - 1 - NSLOT:
                pl.semaphore_signal(
                    credit_f, inc=1,
                    device_id=(left,), device_id_type=pl.DeviceIdType.MESH,
                )
                pl.semaphore_signal(
                    credit_b, inc=1,
                    device_id=(right,), device_id_type=pl.DeviceIdType.MESH,
                )

        for s in range(RT - NSLOT, RT):
            for j in range(KSUB):
                sends_f[(s, j)].wait_send()
                sends_b[(s, j)].wait_send()

    return pl.pallas_call(
        body,
        out_shape=jax.ShapeDtypeStruct((M, N), jnp.float32),
        in_specs=[
            pl.BlockSpec(memory_space=pltpu.SMEM),
            pl.BlockSpec(memory_space=pltpu.SMEM),
            pl.BlockSpec(memory_space=pltpu.SMEM),
            pl.BlockSpec(memory_space=pltpu.VMEM),
            pl.BlockSpec(memory_space=pltpu.VMEM),
        ],
        out_specs=pl.BlockSpec(memory_space=pltpu.VMEM),
        scratch_shapes=[
            pltpu.VMEM((NSUB + KSUB, CHK, H), jnp.float32),
            pltpu.VMEM((NSUB + KSUB, CHK, H), jnp.float32),
            pltpu.VMEM((K, N), jnp.bfloat16),
            pltpu.VMEM((2, CH, H), jnp.float32),
            pltpu.VMEM((2, CH, H), jnp.float32),
            pltpu.SemaphoreType.DMA((NSUB,)),
            pltpu.SemaphoreType.DMA((NSUB,)),
            pltpu.SemaphoreType.DMA((NSUB,)),
            pltpu.SemaphoreType.DMA((NSUB,)),
            pltpu.SemaphoreType.REGULAR,
            pltpu.SemaphoreType.REGULAR,
        ],
        compiler_params=pltpu.CompilerParams(
            collective_id=0,
            vmem_limit_bytes=64 * 1024 * 1024,
        ),
    )(*scalars, A, B)


# device time: 162225 ns/iter; 2.6600x vs baseline; 1.2566x over previous
import jax
import jax.numpy as jnp
from jax import lax
from jax.experimental import pallas as pl
from jax.experimental.pallas import tpu as pltpu

N_DEV = 16
NSLOT = 4
KSUB = 4
RT = 2 * (N_DEV - 1)

RING = [0, 4, 8, 12, 13, 9, 5, 1, 2, 6, 10, 14, 15, 11, 7, 3]
INV_RING = [0] * N_DEV
for _p, _m in enumerate(RING):
    INV_RING[_m] = _p


def kernel(A, B):
    M, K = A.shape
    _, N = B.shape
    CH = M // N_DEV
    H = N // 2
    CHK = CH // KSUB
    NSUB = NSLOT * KSUB
    SEED = NSUB

    ring = jnp.asarray(RING, dtype=jnp.int32)
    inv = jnp.asarray(INV_RING, dtype=jnp.int32)
    m = lax.axis_index("i").astype(jnp.int32)
    r = inv[m]
    right = ring[(r + 1) % N_DEV]
    left = ring[(r - 1) % N_DEV]
    scalars = [jnp.reshape(v, (1,)) for v in (r, left, right)]

    def body(r_ref, left_ref, right_ref, a_ref, b_ref, out_ref,
             comm_f, comm_b, comm_f16, comm_b16, b_bf, pf, pb,
             send_f, recv_f, send_b, recv_b, credit_f, credit_b):
        r = r_ref[0]
        left = left_ref[0]
        right = right_ref[0]

        def stripe(idx, lo, hi):
            a_blk = a_ref[pl.ds(idx * CH, CH), :].astype(jnp.bfloat16)
            return jnp.dot(
                a_blk, b_bf[:, lo:hi], preferred_element_type=jnp.float32
            )

        def sub(s, j):
            return (s % NSLOT) * KSUB + j

        def mk(fwd, s, j):
            if s <= N_DEV - 2:
                comm = comm_f if fwd else comm_b
            else:
                comm = comm_f16 if fwd else comm_b16
            ssem = send_f if fwd else send_b
            rsem = recv_f if fwd else recv_b
            src_idx = SEED + j if s in (0, N_DEV - 1) else sub(s - 1, j)
            return pltpu.make_async_remote_copy(
                src_ref=comm.at[src_idx],
                dst_ref=comm.at[sub(s, j)],
                send_sem=ssem.at[sub(s, j)],
                recv_sem=rsem.at[sub(s, j)],
                device_id=(right if fwd else left,),
                device_id_type=pl.DeviceIdType.MESH,
            )

        b_bf[:, :] = b_ref[:, :].astype(jnp.bfloat16)
        sf = stripe(r, 0, H)
        sb = stripe(r, H, N)
        for j in range(KSUB):
            comm_f[SEED + j, :, :] = sf[j * CHK:(j + 1) * CHK, :]
            comm_b[SEED + j, :, :] = sb[j * CHK:(j + 1) * CHK, :]
        barrier_sem = pltpu.get_barrier_semaphore()
        for nbr in (left, right):
            pl.semaphore_signal(
                barrier_sem, inc=1,
                device_id=(nbr,), device_id_type=pl.DeviceIdType.MESH,
            )
        pl.semaphore_wait(barrier_sem, 2)

        sends_f = {}
        sends_b = {}
        for j in range(KSUB):
            sends_f[(0, j)] = mk(True, 0, j)
            sends_b[(0, j)] = mk(False, 0, j)
            sends_f[(0, j)].start()
            sends_b[(0, j)].start()

        pf[0, :, :] = stripe((r - 1) % N_DEV, 0, H)
        pb[0, :, :] = stripe((r + 1) % N_DEV, H, N)

        for s in range(RT):
            if s + 1 <= N_DEV - 2:
                pf[(s + 1) % 2, :, :] = stripe((r - s - 2) % N_DEV, 0, H)
                pb[(s + 1) % 2, :, :] = stripe((r + s + 2) % N_DEV, H, N)
            for j in range(KSUB):
                mk(True, s, j).wait_recv()
                mk(False, s, j).wait_recv()
                if s <= N_DEV - 2:
                    comm_f[sub(s, j), :, :] = (
                        comm_f[sub(s, j), :, :]
                        + pf[s % 2, pl.ds(j * CHK, CHK), :]
                    )
                    comm_b[sub(s, j), :, :] = (
                        comm_b[sub(s, j), :, :]
                        + pb[s % 2, pl.ds(j * CHK, CHK), :]
                    )
                    if s == N_DEV - 2:
                        comm_f16[SEED + j, :, :] = (
                            comm_f[sub(s, j), :, :].astype(jnp.bfloat16)
                        )
                        comm_b16[SEED + j, :, :] = (
                            comm_b[sub(s, j), :, :].astype(jnp.bfloat16)
                        )
                else:
                    t = s - (N_DEV - 1)
                    idx_f = (r - t) % N_DEV
                    idx_b = (r + t) % N_DEV
                    out_ref[pl.ds(idx_f * CH + j * CHK, CHK), 0:H] = (
                        comm_f16[sub(s, j), :, :].astype(jnp.float32)
                    )
                    out_ref[pl.ds(idx_b * CH + j * CHK, CHK), H:N] = (
                        comm_b16[sub(s, j), :, :].astype(jnp.float32)
                    )
                if s < RT - 1:
                    if j == 0 and s + 1 >= NSLOT:
                        pl.semaphore_wait(credit_f, 1)
                        pl.semaphore_wait(credit_b, 1)
                    if s + 1 >= NSLOT:
                        sends_f[(s + 1 - NSLOT, j)].wait_send()
                        sends_b[(s + 1 - NSLOT, j)].wait_send()
                    nf = mk(True, s + 1, j)
                    nb = mk(False, s + 1, j)
                    nf.start()
                    nb.start()
                    sends_f[(s + 1, j)] = nf
                    sends_b[(s + 1, j)] = nb
            if s == N_DEV - 2:
                own_f = (r + 1) % N_DEV
                own_b = (r - 1) % N_DEV
                for j in range(KSUB):
                    out_ref[pl.ds(own_f * CH + j * CHK, CHK), 0:H] = (
                        comm_f[sub(s, j), :, :]
                    )
                    out_ref[pl.ds(own_b * CH + j * CHK, CHK), H:N] = (
                        comm_b[sub(s, j), :, :]
                    )
            if s <= RT - 1 - NSLOT:
                pl.semaphore_signal(
                    credit_f, inc=1,
                    device_id=(left,), device_id_type=pl.DeviceIdType.MESH,
                )
                pl.semaphore_signal(
                    credit_b, inc=1,
                    device_id=(right,), device_id_type=pl.DeviceIdType.MESH,
                )

        for s in range(RT - NSLOT, RT):
            for j in range(KSUB):
                sends_f[(s, j)].wait_send()
                sends_b[(s, j)].wait_send()

    return pl.pallas_call(
        body,
        out_shape=jax.ShapeDtypeStruct((M, N), jnp.float32),
        in_specs=[
            pl.BlockSpec(memory_space=pltpu.SMEM),
            pl.BlockSpec(memory_space=pltpu.SMEM),
            pl.BlockSpec(memory_space=pltpu.SMEM),
            pl.BlockSpec(memory_space=pltpu.VMEM),
            pl.BlockSpec(memory_space=pltpu.VMEM),
        ],
        out_specs=pl.BlockSpec(memory_space=pltpu.VMEM),
        scratch_shapes=[
            pltpu.VMEM((NSUB + KSUB, CHK, H), jnp.float32),
            pltpu.VMEM((NSUB + KSUB, CHK, H), jnp.float32),
            pltpu.VMEM((NSUB + KSUB, CHK, H), jnp.bfloat16),
            pltpu.VMEM((NSUB + KSUB, CHK, H), jnp.bfloat16),
            pltpu.VMEM((K, N), jnp.bfloat16),
            pltpu.VMEM((2, CH, H), jnp.float32),
            pltpu.VMEM((2, CH, H), jnp.float32),
            pltpu.SemaphoreType.DMA((NSUB,)),
            pltpu.SemaphoreType.DMA((NSUB,)),
            pltpu.SemaphoreType.DMA((NSUB,)),
            pltpu.SemaphoreType.DMA((NSUB,)),
            pltpu.SemaphoreType.REGULAR,
            pltpu.SemaphoreType.REGULAR,
        ],
        compiler_params=pltpu.CompilerParams(
            collective_id=0,
            vmem_limit_bytes=64 * 1024 * 1024,
        ),
    )(*scalars, A, B)


# device time: 120649 ns/iter; 3.5767x vs baseline; 1.3446x over previous
import jax
import jax.numpy as jnp
from jax import lax
from jax.experimental import pallas as pl
from jax.experimental.pallas import tpu as pltpu

N_DEV = 16
NSLOT = 4
KSUB = 4
RT = 2 * (N_DEV - 1)

RING = [0, 4, 8, 12, 13, 9, 5, 1, 2, 6, 10, 14, 15, 11, 7, 3]
INV_RING = [0] * N_DEV
for _p, _m in enumerate(RING):
    INV_RING[_m] = _p


def kernel(A, B):
    M, K = A.shape
    _, N = B.shape
    CH = M // N_DEV
    H = N // 2
    CHK = CH // KSUB
    NSUB = NSLOT * KSUB
    SEED = NSUB

    ring = jnp.asarray(RING, dtype=jnp.int32)
    inv = jnp.asarray(INV_RING, dtype=jnp.int32)
    m = lax.axis_index("i").astype(jnp.int32)
    r = inv[m]
    right = ring[(r + 1) % N_DEV]
    left = ring[(r - 1) % N_DEV]
    scalars = [jnp.reshape(v, (1,)) for v in (r, left, right)]

    def body(r_ref, left_ref, right_ref, a_ref, b_ref, out_ref,
             comm_f, comm_b, b_bf, pf, pb,
             send_f, recv_f, send_b, recv_b, credit_f, credit_b):
        r = r_ref[0]
        left = left_ref[0]
        right = right_ref[0]

        def stripe(idx, lo, hi):
            a_blk = a_ref[pl.ds(idx * CH, CH), :].astype(jnp.bfloat16)
            return jnp.dot(
                a_blk, b_bf[:, lo:hi], preferred_element_type=jnp.float32
            )

        def sub(s, j):
            return (s % NSLOT) * KSUB + j

        def mk(fwd, s, j):
            comm = comm_f if fwd else comm_b
            ssem = send_f if fwd else send_b
            rsem = recv_f if fwd else recv_b
            src_idx = SEED + j if s == 0 else sub(s - 1, j)
            return pltpu.make_async_remote_copy(
                src_ref=comm.at[src_idx],
                dst_ref=comm.at[sub(s, j)],
                send_sem=ssem.at[sub(s, j)],
                recv_sem=rsem.at[sub(s, j)],
                device_id=(right if fwd else left,),
                device_id_type=pl.DeviceIdType.MESH,
            )

        b_bf[:, :] = b_ref[:, :].astype(jnp.bfloat16)
        sf = stripe(r, 0, H)
        sb = stripe(r, H, N)
        for j in range(KSUB):
            comm_f[SEED + j, :, :] = (
                sf[j * CHK:(j + 1) * CHK, :].astype(jnp.bfloat16)
            )
            comm_b[SEED + j, :, :] = (
                sb[j * CHK:(j + 1) * CHK, :].astype(jnp.bfloat16)
            )
        barrier_sem = pltpu.get_barrier_semaphore()
        for nbr in (left, right):
            pl.semaphore_signal(
                barrier_sem, inc=1,
                device_id=(nbr,), device_id_type=pl.DeviceIdType.MESH,
            )
        pl.semaphore_wait(barrier_sem, 2)

        sends_f = {}
        sends_b = {}
        for j in range(KSUB):
            sends_f[(0, j)] = mk(True, 0, j)
            sends_b[(0, j)] = mk(False, 0, j)
            sends_f[(0, j)].start()
            sends_b[(0, j)].start()

        pf[0, :, :] = stripe((r - 1) % N_DEV, 0, H)
        pb[0, :, :] = stripe((r + 1) % N_DEV, H, N)

        for s in range(RT):
            if s + 1 <= N_DEV - 2:
                pf[(s + 1) % 2, :, :] = stripe((r - s - 2) % N_DEV, 0, H)
                pb[(s + 1) % 2, :, :] = stripe((r + s + 2) % N_DEV, H, N)
            for j in range(KSUB):
                mk(True, s, j).wait_recv()
                mk(False, s, j).wait_recv()
                if s <= N_DEV - 2:
                    comm_f[sub(s, j), :, :] = (
                        comm_f[sub(s, j), :, :].astype(jnp.float32)
                        + pf[s % 2, pl.ds(j * CHK, CHK), :]
                    ).astype(jnp.bfloat16)
                    comm_b[sub(s, j), :, :] = (
                        comm_b[sub(s, j), :, :].astype(jnp.float32)
                        + pb[s % 2, pl.ds(j * CHK, CHK), :]
                    ).astype(jnp.bfloat16)
                else:
                    t = s - (N_DEV - 1)
                    idx_f = (r - t) % N_DEV
                    idx_b = (r + t) % N_DEV
                    out_ref[pl.ds(idx_f * CH + j * CHK, CHK), 0:H] = (
                        comm_f[sub(s, j), :, :].astype(jnp.float32)
                    )
                    out_ref[pl.ds(idx_b * CH + j * CHK, CHK), H:N] = (
                        comm_b[sub(s, j), :, :].astype(jnp.float32)
                    )
                if s < RT - 1:
                    if j == 0 and s + 1 >= NSLOT:
                        pl.semaphore_wait(credit_f, 1)
                        pl.semaphore_wait(credit_b, 1)
                    if s + 1 >= NSLOT:
                        sends_f[(s + 1 - NSLOT, j)].wait_send()
                        sends_b[(s + 1 - NSLOT, j)].wait_send()
                    nf = mk(True, s + 1, j)
                    nb = mk(False, s + 1, j)
                    nf.start()
                    nb.start()
                    sends_f[(s + 1, j)] = nf
                    sends_b[(s + 1, j)] = nb
            if s == N_DEV - 2:
                own_f = (r + 1) % N_DEV
                own_b = (r - 1) % N_DEV
                for j in range(KSUB):
                    out_ref[pl.ds(own_f * CH + j * CHK, CHK), 0:H] = (
                        comm_f[sub(s, j), :, :].astype(jnp.float32)
                    )
                    out_ref[pl.ds(own_b * CH + j * CHK, CHK), H:N] = (
                        comm_b[sub(s, j), :, :].astype(jnp.float32)
                    )
            if s <= RT - 1 - NSLOT:
                pl.semaphore_signal(
                    credit_f, inc=1,
                    device_id=(left,), device_id_type=pl.DeviceIdType.MESH,
                )
                pl.semaphore_signal(
                    credit_b, inc=1,
                    device_id=(right,), device_id_type=pl.DeviceIdType.MESH,
                )

        for s in range(RT - NSLOT, RT):
            for j in range(KSUB):
                sends_f[(s, j)].wait_send()
                sends_b[(s, j)].wait_send()

    return pl.pallas_call(
        body,
        out_shape=jax.ShapeDtypeStruct((M, N), jnp.float32),
        in_specs=[
            pl.BlockSpec(memory_space=pltpu.SMEM),
            pl.BlockSpec(memory_space=pltpu.SMEM),
            pl.BlockSpec(memory_space=pltpu.SMEM),
            pl.BlockSpec(memory_space=pltpu.VMEM),
            pl.BlockSpec(memory_space=pltpu.VMEM),
        ],
        out_specs=pl.BlockSpec(memory_space=pltpu.VMEM),
        scratch_shapes=[
            pltpu.VMEM((NSUB + KSUB, CHK, H), jnp.bfloat16),
            pltpu.VMEM((NSUB + KSUB, CHK, H), jnp.bfloat16),
            pltpu.VMEM((K, N), jnp.bfloat16),
            pltpu.VMEM((2, CH, H), jnp.float32),
            pltpu.VMEM((2, CH, H), jnp.float32),
            pltpu.SemaphoreType.DMA((NSUB,)),
            pltpu.SemaphoreType.DMA((NSUB,)),
            pltpu.SemaphoreType.DMA((NSUB,)),
            pltpu.SemaphoreType.DMA((NSUB,)),
            pltpu.SemaphoreType.REGULAR,
            pltpu.SemaphoreType.REGULAR,
        ],
        compiler_params=pltpu.CompilerParams(
            collective_id=0,
            vmem_limit_bytes=64 * 1024 * 1024,
        ),
    )(*scalars, A, B)
